# baseline (speedup 1.0000x reference)
import functools

import jax
import jax.numpy as jnp
from jax import lax
from jax.experimental import pallas as pl
from jax.experimental.pallas import tpu as pltpu
from jax.experimental.pallas import tpu_sc as plsc

N_DST = 10000
D = 128
NC = 2
NS = 16
NW = NC * NS
CHUNK = 80
M = 128
NBUF = 4
E_PAD = NW * M * CHUNK
ACC_ROWS = 10240
ROWS_PER_TILE = ACC_ROWS // NS


def _sc_partial_sums(node, src, dst, zeros):
    mesh = plsc.VectorSubcoreMesh(core_axis_name="c", subcore_axis_name="s")

    @functools.partial(
        pl.kernel,
        mesh=mesh,
        out_type=jax.ShapeDtypeStruct((NC * ACC_ROWS, D), jnp.float32),
        scratch_types=[
            *[pltpu.VMEM((CHUNK,), jnp.int32) for _ in range(NBUF)],
            *[pltpu.VMEM((CHUNK,), jnp.int32) for _ in range(NBUF)],
            *[pltpu.VMEM((CHUNK, D), jnp.float32) for _ in range(NBUF)],
            pltpu.VMEM_SHARED((ACC_ROWS, D), jnp.float32),
            *[pltpu.SemaphoreType.DMA for _ in range(3 * NBUF)],
        ],
    )
    def k(node_hbm, src_hbm, dst_hbm, zeros_hbm, out_hbm, *scr):
        src_v = scr[0:NBUF]
        dst_v = scr[NBUF:2 * NBUF]
        rows_v = scr[2 * NBUF:3 * NBUF]
        acc = scr[3 * NBUF]
        sem_i = scr[3 * NBUF + 1:3 * NBUF + 1 + NBUF]
        sem_g = scr[3 * NBUF + 1 + NBUF:3 * NBUF + 1 + 2 * NBUF]
        sem_s = scr[3 * NBUF + 1 + 2 * NBUF:3 * NBUF + 1 + 3 * NBUF]

        c = lax.axis_index("c")
        s = lax.axis_index("s")
        wid = s * NC + c

        r0 = s * ROWS_PER_TILE
        pltpu.sync_copy(zeros_hbm.at[pl.ds(r0, ROWS_PER_TILE)],
                        acc.at[pl.ds(r0, ROWS_PER_TILE)])
        plsc.subcore_barrier()

        base0 = wid * (M * CHUNK)

        def idx_start(chunk, b):
            e = base0 + chunk * CHUNK
            pltpu.async_copy(src_hbm.at[pl.ds(e, CHUNK)], src_v[b], sem_i[b])
            pltpu.async_copy(dst_hbm.at[pl.ds(e, CHUNK)], dst_v[b], sem_i[b])

        def idx_wait(chunk, b):
            e = base0 + chunk * CHUNK
            pltpu.make_async_copy(src_hbm.at[pl.ds(e, CHUNK)], src_v[b],
                                  sem_i[b]).wait()
            pltpu.make_async_copy(dst_hbm.at[pl.ds(e, CHUNK)], dst_v[b],
                                  sem_i[b]).wait()

        def gather_start(b):
            pltpu.async_copy(node_hbm.at[src_v[b]], rows_v[b], sem_g[b])

        def gather_wait(b):
            pltpu.make_async_copy(node_hbm.at[src_v[b]], rows_v[b],
                                  sem_g[b]).wait()

        def scatter_start(b):
            pltpu.async_copy(rows_v[b], acc.at[dst_v[b]], sem_s[b], add=True)

        def scatter_wait(b):
            pltpu.make_async_copy(rows_v[b], acc.at[dst_v[b]], sem_s[b]).wait()

        for b in range(NBUF):
            idx_start(b, b)
        for b in range(NBUF):
            idx_wait(b, b)
            gather_start(b)

        n_outer = M // NBUF

        def outer(g, carry):
            for b in range(NBUF):
                gather_wait(b)
                scatter_start(b)
            for b in range(NBUF):
                @pl.when(g + 1 < n_outer)
                def _():
                    chunk = g * NBUF + b
                    scatter_wait(b)
                    idx_start(chunk + NBUF, b)
            for b in range(NBUF):
                @pl.when(g + 1 < n_outer)
                def _():
                    chunk = g * NBUF + b
                    idx_wait(chunk + NBUF, b)
                    gather_start(b)
            return carry

        lax.fori_loop(0, n_outer, outer, 0)
        for b in range(NBUF):
            scatter_wait(b)
        plsc.subcore_barrier()

        pltpu.sync_copy(acc.at[pl.ds(r0, ROWS_PER_TILE)],
                        out_hbm.at[pl.ds(c * ACC_ROWS + r0, ROWS_PER_TILE)])

    return k(node, src, dst, zeros)


def _combine(partials):
    R = 400

    def body(a_ref, b_ref, o_ref):
        o_ref[...] = a_ref[...] + b_ref[...]

    return pl.pallas_call(
        body,
        grid=(N_DST // R,),
        in_specs=[pl.BlockSpec((R, D), lambda i: (i, 0)),
                  pl.BlockSpec((R, D), lambda i: (i, 0))],
        out_specs=pl.BlockSpec((R, D), lambda i: (i, 0)),
        out_shape=jax.ShapeDtypeStruct((N_DST, D), jnp.float32),
    )(partials[:N_DST], partials[ACC_ROWS:ACC_ROWS + N_DST])


def kernel(node, edge_index):
    ei = edge_index.astype(jnp.int32)
    E = ei.shape[1]
    pad = E_PAD - E
    src = jnp.concatenate([ei[0], jnp.zeros((pad,), jnp.int32)])
    dst = jnp.concatenate([ei[1], jnp.full((pad,), N_DST, jnp.int32)])
    zeros = jnp.zeros((ACC_ROWS, D), jnp.float32)
    partials = _sc_partial_sums(node, src, dst, zeros)
    return _combine(partials)


# speedup vs baseline: 1.0338x; 1.0338x over previous
import functools

import jax
import jax.numpy as jnp
from jax import lax
from jax.experimental import pallas as pl
from jax.experimental.pallas import tpu as pltpu
from jax.experimental.pallas import tpu_sc as plsc

N_DST = 10000
D = 128
NC = 2
NS = 16
NW = NC * NS
CHUNK = 80
BLK = 8
NBLK = 16
M = BLK * NBLK
E_PAD = NW * M * CHUNK
ACC_ROWS = 10240
ROWS_PER_TILE = ACC_ROWS // NS


def _sc_partial_sums(node, src4, dst4, zeros):
    mesh = plsc.VectorSubcoreMesh(core_axis_name="c", subcore_axis_name="s")

    @functools.partial(
        pl.kernel,
        mesh=mesh,
        out_type=jax.ShapeDtypeStruct((NC * ACC_ROWS, D), jnp.float32),
        scratch_types=[
            pltpu.VMEM((BLK, CHUNK), jnp.int32),
            pltpu.VMEM((BLK, CHUNK), jnp.int32),
            pltpu.VMEM((BLK, CHUNK), jnp.int32),
            pltpu.VMEM((BLK, CHUNK), jnp.int32),
            pltpu.VMEM((CHUNK, D), jnp.float32),
            pltpu.VMEM((CHUNK, D), jnp.float32),
            pltpu.VMEM_SHARED((ACC_ROWS, D), jnp.float32),
            pltpu.SemaphoreType.DMA,
            pltpu.SemaphoreType.DMA,
            pltpu.SemaphoreType.DMA,
            pltpu.SemaphoreType.DMA,
            pltpu.SemaphoreType.DMA,
        ],
    )
    def k(node_hbm, src_hbm, dst_hbm, zeros_hbm, out_hbm,
          src_b0, src_b1, dst_b0, dst_b1, rows_0, rows_1, acc,
          sem_i0, sem_i1, sem_g0, sem_g1, sem_s):
        src_blk = [src_b0, src_b1]
        dst_blk = [dst_b0, dst_b1]
        rows = [rows_0, rows_1]
        sem_i = [sem_i0, sem_i1]
        sem_g = [sem_g0, sem_g1]

        c = lax.axis_index("c")
        s = lax.axis_index("s")
        wid = s * NC + c

        r0 = s * ROWS_PER_TILE
        pltpu.sync_copy(zeros_hbm.at[pl.ds(r0, ROWS_PER_TILE)],
                        acc.at[pl.ds(r0, ROWS_PER_TILE)])
        plsc.subcore_barrier()

        def blk_start(bidx, p):
            pltpu.async_copy(src_hbm.at[wid, bidx], src_blk[p], sem_i[p])
            pltpu.async_copy(dst_hbm.at[wid, bidx], dst_blk[p], sem_i[p])

        def blk_wait(bidx, p):
            pltpu.make_async_copy(src_hbm.at[wid, bidx], src_blk[p],
                                  sem_i[p]).wait()
            pltpu.make_async_copy(dst_hbm.at[wid, bidx], dst_blk[p],
                                  sem_i[p]).wait()

        def gather_start(p, j, b):
            pltpu.async_copy(node_hbm.at[src_blk[p].at[j]], rows[b], sem_g[b])

        def gather_wait(p, j, b):
            pltpu.make_async_copy(node_hbm.at[src_blk[p].at[j]], rows[b],
                                  sem_g[b]).wait()

        def scatter_sync(p, j, b):
            pltpu.async_copy(rows[b], acc.at[dst_blk[p].at[j]], sem_s,
                             add=True).wait()

        blk_start(0, 0)
        blk_wait(0, 0)
        gather_start(0, 0, 0)

        def outer(gg, carry):
            for gp in range(2):
                bidx = 2 * gg + gp
                bnext = bidx + 1
                if gp == 0:
                    blk_start(bnext, 1)
                else:
                    @pl.when(gg < NBLK // 2 - 1)
                    def _():
                        blk_start(bnext, 0)
                for j in range(BLK):
                    b = j % 2
                    if j < BLK - 1:
                        gather_start(gp, j + 1, b ^ 1)
                    elif gp == 0:
                        blk_wait(bnext, 1)
                        gather_start(1, 0, b ^ 1)
                    else:
                        @pl.when(gg < NBLK // 2 - 1)
                        def _():
                            blk_wait(bnext, 0)
                            gather_start(0, 0, b ^ 1)
                    gather_wait(gp, j, b)
                    scatter_sync(gp, j, b)
            return carry

        lax.fori_loop(0, NBLK // 2, outer, 0)
        plsc.subcore_barrier()

        pltpu.sync_copy(acc.at[pl.ds(r0, ROWS_PER_TILE)],
                        out_hbm.at[pl.ds(c * ACC_ROWS + r0, ROWS_PER_TILE)])

    return k(node, src4, dst4, zeros)


def _combine(partials):
    R = 400

    def body(a_ref, b_ref, o_ref):
        o_ref[...] = a_ref[...] + b_ref[...]

    return pl.pallas_call(
        body,
        grid=(N_DST // R,),
        in_specs=[pl.BlockSpec((R, D), lambda i: (i, 0)),
                  pl.BlockSpec((R, D), lambda i: (i, 0))],
        out_specs=pl.BlockSpec((R, D), lambda i: (i, 0)),
        out_shape=jax.ShapeDtypeStruct((N_DST, D), jnp.float32),
    )(partials[:N_DST], partials[ACC_ROWS:ACC_ROWS + N_DST])


def kernel(node, edge_index):
    ei = edge_index.astype(jnp.int32)
    E = ei.shape[1]
    pad = E_PAD - E
    src = jnp.concatenate([ei[0], jnp.zeros((pad,), jnp.int32)])
    dst = jnp.concatenate([ei[1], jnp.full((pad,), N_DST, jnp.int32)])
    src4 = src.reshape(NW, NBLK, BLK, CHUNK)
    dst4 = dst.reshape(NW, NBLK, BLK, CHUNK)
    zeros = jnp.zeros((ACC_ROWS, D), jnp.float32)
    partials = _sc_partial_sums(node, src4, dst4, zeros)
    return _combine(partials)
